# baseline (speedup 1.0000x reference)
import functools

import jax
import jax.numpy as jnp
from jax import lax
from jax.experimental import pallas as pl
from jax.experimental.pallas import tpu as pltpu
from jax.experimental.pallas import tpu_sc as plsc

B = 16
AB = 20000
V = 12000
NW = 32
NJ = 3
G = 128
N = NJ * G
LASTN = V - (NW - 1) * N
W = 768
NP = 2
AROW = 5 * B


def _sc_body(planes_hbm, idx_hbm, out_hbm,
             idx_v, loc_v, ax_v, ay_v, aw_v, ah_v,
             wd_v, os_v, ox_v, oy_v, ow_v, oh_v,
             *sems):
    sem_g = sems[:NP]
    sem_s = sems[NP:]
    wid = lax.axis_index("s") * 2 + lax.axis_index("c")
    base = wid * N
    is_last = wid == NW - 1
    not_last = wid != NW - 1

    @pl.when(not_last)
    def _():
        for j in range(NJ):
            pltpu.sync_copy(idx_hbm.at[pl.ds(base + G * j, G)], idx_v.at[j])

    @pl.when(is_last)
    def _():
        pltpu.sync_copy(idx_hbm.at[pl.ds(base, LASTN)],
                        idx_v.at[0].at[pl.ds(0, LASTN)])
        fill = jnp.broadcast_to(
            lax.reduce_max(idx_v[0, pl.ds(LASTN - 16, 16)], (0,)), (16,))
        for i in range(LASTN // 16, G // 16):
            idx_v[0, pl.ds(i * 16, 16)] = fill
        for j in range(1, NJ):
            for i in range(G // 16):
                idx_v[j, pl.ds(i * 16, 16)] = fill

    cps = []
    for j in range(NJ):
        ij = idx_v.at[j]
        cps.append(pltpu.async_copy(
            planes_hbm.at[AROW].at[ij], ax_v.at[j], sem_g[0]))
        cps.append(pltpu.async_copy(
            planes_hbm.at[AROW + 1].at[ij], ay_v.at[j], sem_g[0]))
        cps.append(pltpu.async_copy(
            planes_hbm.at[AROW + 2].at[ij], aw_v.at[j], sem_g[0]))
        cps.append(pltpu.async_copy(
            planes_hbm.at[AROW + 3].at[ij], ah_v.at[j], sem_g[0]))
    for cp in cps:
        cp.wait()

    lo = lax.reduce_min(idx_v[0, pl.ds(0, 16)], (0,))
    hi = lax.reduce_max(idx_v[NJ - 1, pl.ds(G - 16, 16)], (0,))
    lo_al = pl.multiple_of(
        jnp.minimum((lo >> 3) << 3, jnp.int32(AB - W)), 8)
    span_ok = (hi - lo_al) < W
    span_bad = jnp.logical_not(span_ok)
    iota = lax.iota(jnp.int32, 16)
    for j in range(NJ):
        for i in range(G // 16):
            sl = pl.ds(i * 16, 16)
            flat = iota + (j * G + i * 16)
            loc_v[j, sl] = jnp.where(span_ok, idx_v[j, sl] - lo_al, flat)

    def fast_descs(b, p):
        return [
            pltpu.make_async_copy(
                planes_hbm.at[pl.ds(b * 5, 5), pl.ds(lo_al, W)],
                wd_v.at[p], sem_g[p]),
        ]

    def slow_descs(b, p):
        ds = []
        for j in range(NJ):
            ij = idx_v.at[j]
            gsl = pl.ds(j * G, G)
            for c in range(5):
                ds.append(pltpu.make_async_copy(
                    planes_hbm.at[b * 5 + c].at[ij],
                    wd_v.at[p].at[c].at[gsl], sem_g[p]))
        return ds

    def fire_gathers(b, p):
        @pl.when(span_ok)
        def _():
            for d in fast_descs(b, p):
                d.start()

        @pl.when(span_bad)
        def _():
            for d in slow_descs(b, p):
                d.start()

    def wait_gathers(b, p):
        @pl.when(span_ok)
        def _():
            for d in fast_descs(b, p):
                d.wait()

        @pl.when(span_bad)
        def _():
            for d in slow_descs(b, p):
                d.wait()

    def store_descs(b, p):
        ds = []
        for c, buf in ((0, os_v), (1, ox_v), (2, oy_v), (3, ow_v), (4, oh_v)):
            ds.append((0, pltpu.make_async_copy(
                buf.at[p], out_hbm.at[c].at[b].at[pl.ds(base, N)], sem_s[p])))
            ds.append((-1, pltpu.make_async_copy(
                buf.at[p].at[pl.ds(0, LASTN)],
                out_hbm.at[c].at[b].at[pl.ds(base, LASTN)], sem_s[p])))
        return ds

    def fire_stores(b, p):
        for j, d in store_descs(b, p):
            if j >= 0:
                @pl.when(not_last)
                def _():
                    d.start()
            else:
                @pl.when(is_last)
                def _():
                    d.start()

    def drain_stores(b, p):
        for j, d in store_descs(b, p):
            if j >= 0:
                @pl.when(not_last)
                def _():
                    d.wait()
            else:
                @pl.when(is_last)
                def _():
                    d.wait()

    def compute(p):
        for j in range(NJ):
            for i in range(G // 16):
                sl = pl.ds(i * 16, 16)
                slo = pl.ds(j * G + i * 16, 16)
                loc = loc_v[j, sl]
                s = plsc.load_gather(wd_v.at[p].at[0], [loc])
                dx = plsc.load_gather(wd_v.at[p].at[1], [loc])
                dy = plsc.load_gather(wd_v.at[p].at[2], [loc])
                dw = plsc.load_gather(wd_v.at[p].at[3], [loc])
                dh = plsc.load_gather(wd_v.at[p].at[4], [loc])
                ax = ax_v[j, sl]
                ay = ay_v[j, sl]
                aw = aw_v[j, sl]
                ah = ah_v[j, sl]
                os_v[p, slo] = s
                ox_v[p, slo] = ax + dx * aw
                oy_v[p, slo] = ay + dy * ah
                ow_v[p, slo] = aw * jnp.exp(dw)
                oh_v[p, slo] = ah * jnp.exp(dh)

    for b in range(NP - 1):
        fire_gathers(b, b)

    def body(t, carry):
        for k in range(NP):
            b = NP * t + k
            nb = b + NP - 1
            kb = (k + NP - 1) % NP

            @pl.when(nb < B)
            def _():
                fire_gathers(nb, kb)
            wait_gathers(b, k)

            @pl.when(t > 0)
            def _():
                drain_stores(b - NP, k)
            compute(k)
            fire_stores(b, k)
        return carry

    lax.fori_loop(0, B // NP, body, 0)
    for b in range(B - NP, B):
        drain_stores(b, b % NP)


_PLANE = pltpu.VMEM((NJ, G), jnp.float32)
_FLATP = pltpu.VMEM((NP, N), jnp.float32)


@functools.partial(
    pl.kernel,
    out_type=jax.ShapeDtypeStruct((5, B, V), jnp.float32),
    mesh=plsc.VectorSubcoreMesh(core_axis_name="c", subcore_axis_name="s"),
    compiler_params=pltpu.CompilerParams(
        needs_layout_passes=False, use_tc_tiling_on_sc=False),
    scratch_types=[
        pltpu.VMEM((NJ, G), jnp.int32),
        pltpu.VMEM((NJ, G), jnp.int32),
        _PLANE, _PLANE, _PLANE, _PLANE,
        pltpu.VMEM((NP, 5, W), jnp.float32),
        _FLATP, _FLATP, _FLATP, _FLATP, _FLATP,
    ] + [pltpu.SemaphoreType.DMA] * (2 * NP),
)
def _apply_deltas_sc(planes_hbm, idx_hbm, out_hbm, *refs):
    _sc_body(planes_hbm, idx_hbm, out_hbm, *refs)


def kernel(scores, deltas, anchor_boxes, valid_indices):
    idx = valid_indices.astype(jnp.int32)
    batch_planes = jnp.concatenate(
        [scores[:, None, :], jnp.transpose(deltas, (0, 2, 1))], axis=1)
    planes = jnp.concatenate(
        [batch_planes.reshape(5 * B, AB), jnp.transpose(anchor_boxes, (1, 0))],
        axis=0)
    out = _apply_deltas_sc(planes, idx)
    return jnp.transpose(out, (1, 2, 0))


# speedup vs baseline: 1.3461x; 1.3461x over previous
import functools

import jax
import jax.numpy as jnp
from jax import lax
from jax.experimental import pallas as pl
from jax.experimental.pallas import tpu as pltpu
from jax.experimental.pallas import tpu_sc as plsc

B = 16
AB = 20000
V = 12000
NW = 32
NJ = 3
G = 128
N = NJ * G
LASTN = V - (NW - 1) * N
W = 768
NP = 2


def _sc_body(scores_hbm, deltas_hbm, anch_hbm, idx_hbm, out_hbm,
             idx_v, loc_v, ax_v, ay_v, aw_v, ah_v,
             ws_v, wd_v,
             os_v, ox_v, oy_v, ow_v, oh_v,
             *sems):
    sem_g = sems[:NP]
    sem_s = sems[NP:]
    wid = lax.axis_index("s") * 2 + lax.axis_index("c")
    base = wid * N
    is_last = wid == NW - 1
    not_last = wid != NW - 1

    @pl.when(not_last)
    def _():
        for j in range(NJ):
            pltpu.sync_copy(idx_hbm.at[pl.ds(base + G * j, G)], idx_v.at[j])

    @pl.when(is_last)
    def _():
        pltpu.sync_copy(idx_hbm.at[pl.ds(base, LASTN)],
                        idx_v.at[0].at[pl.ds(0, LASTN)])
        fill = jnp.broadcast_to(
            lax.reduce_max(idx_v[0, pl.ds(LASTN - 16, 16)], (0,)), (16,))
        for i in range(LASTN // 16, G // 16):
            idx_v[0, pl.ds(i * 16, 16)] = fill
        for j in range(1, NJ):
            for i in range(G // 16):
                idx_v[j, pl.ds(i * 16, 16)] = fill

    cps = []
    for j in range(NJ):
        ij = idx_v.at[j]
        cps.append(pltpu.async_copy(anch_hbm.at[0].at[ij], ax_v.at[j], sem_g[0]))
        cps.append(pltpu.async_copy(anch_hbm.at[1].at[ij], ay_v.at[j], sem_g[0]))
        cps.append(pltpu.async_copy(anch_hbm.at[2].at[ij], aw_v.at[j], sem_g[0]))
        cps.append(pltpu.async_copy(anch_hbm.at[3].at[ij], ah_v.at[j], sem_g[0]))
    for cp in cps:
        cp.wait()

    lo = lax.reduce_min(idx_v[0, pl.ds(0, 16)], (0,))
    hi = lax.reduce_max(idx_v[NJ - 1, pl.ds(G - 16, 16)], (0,))
    lo_al = pl.multiple_of(
        jnp.minimum((lo >> 3) << 3, jnp.int32(AB - W)), 8)
    span_ok = (hi - lo_al) < W
    span_bad = jnp.logical_not(span_ok)
    iota = lax.iota(jnp.int32, 16)
    for j in range(NJ):
        for i in range(G // 16):
            sl = pl.ds(i * 16, 16)
            flat = iota + (j * G + i * 16)
            loc_v[j, sl] = jnp.where(span_ok, idx_v[j, sl] - lo_al, flat)

    def fast_descs(b, p):
        row = b * 4
        win = pl.ds(lo_al, W)
        return [
            pltpu.make_async_copy(
                scores_hbm.at[b].at[win], ws_v.at[p], sem_g[p]),
            pltpu.make_async_copy(
                deltas_hbm.at[pl.ds(row, 4), win], wd_v.at[p], sem_g[p]),
        ]

    def slow_descs(b, p):
        row = b * 4
        ds = []
        for j in range(NJ):
            ij = idx_v.at[j]
            gsl = pl.ds(j * G, G)
            ds.append(pltpu.make_async_copy(
                scores_hbm.at[b].at[ij], ws_v.at[p].at[gsl], sem_g[p]))
            for c in range(4):
                ds.append(pltpu.make_async_copy(
                    deltas_hbm.at[row + c].at[ij],
                    wd_v.at[p].at[c].at[gsl], sem_g[p]))
        return ds

    def fire_gathers(b, p):
        @pl.when(span_ok)
        def _():
            for d in fast_descs(b, p):
                d.start()

        @pl.when(span_bad)
        def _():
            for d in slow_descs(b, p):
                d.start()

    def wait_gathers(b, p):
        @pl.when(span_ok)
        def _():
            for d in fast_descs(b, p):
                d.wait()

        @pl.when(span_bad)
        def _():
            for d in slow_descs(b, p):
                d.wait()

    def store_descs(b, p):
        ds = []
        for c, buf in ((0, os_v), (1, ox_v), (2, oy_v), (3, ow_v), (4, oh_v)):
            ds.append((0, pltpu.make_async_copy(
                buf.at[p], out_hbm.at[c].at[b].at[pl.ds(base, N)], sem_s[p])))
            ds.append((-1, pltpu.make_async_copy(
                buf.at[p].at[pl.ds(0, LASTN)],
                out_hbm.at[c].at[b].at[pl.ds(base, LASTN)], sem_s[p])))
        return ds

    def fire_stores(b, p):
        for j, d in store_descs(b, p):
            if j >= 0:
                @pl.when(not_last)
                def _():
                    d.start()
            else:
                @pl.when(is_last)
                def _():
                    d.start()

    def drain_stores(b, p):
        for j, d in store_descs(b, p):
            if j >= 0:
                @pl.when(not_last)
                def _():
                    d.wait()
            else:
                @pl.when(is_last)
                def _():
                    d.wait()

    def compute(p):
        for j in range(NJ):
            for i in range(G // 16):
                sl = pl.ds(i * 16, 16)
                slo = pl.ds(j * G + i * 16, 16)
                loc = loc_v[j, sl]
                s = plsc.load_gather(ws_v.at[p], [loc])
                dx = plsc.load_gather(wd_v.at[p].at[0], [loc])
                dy = plsc.load_gather(wd_v.at[p].at[1], [loc])
                dw = plsc.load_gather(wd_v.at[p].at[2], [loc])
                dh = plsc.load_gather(wd_v.at[p].at[3], [loc])
                ax = ax_v[j, sl]
                ay = ay_v[j, sl]
                aw = aw_v[j, sl]
                ah = ah_v[j, sl]
                os_v[p, slo] = s
                ox_v[p, slo] = ax + dx * aw
                oy_v[p, slo] = ay + dy * ah
                ow_v[p, slo] = aw * jnp.exp(dw)
                oh_v[p, slo] = ah * jnp.exp(dh)

    for b in range(NP - 1):
        fire_gathers(b, b)

    def body(t, carry):
        for k in range(NP):
            b = NP * t + k
            nb = b + NP - 1
            kb = (k + NP - 1) % NP

            @pl.when(nb < B)
            def _():
                fire_gathers(nb, kb)
            wait_gathers(b, k)

            @pl.when(t > 0)
            def _():
                drain_stores(b - NP, k)
            compute(k)
            fire_stores(b, k)
        return carry

    lax.fori_loop(0, B // NP, body, 0)
    for b in range(B - NP, B):
        drain_stores(b, b % NP)


_PLANE = pltpu.VMEM((NJ, G), jnp.float32)
_FLATP = pltpu.VMEM((NP, N), jnp.float32)


@functools.partial(
    pl.kernel,
    out_type=jax.ShapeDtypeStruct((5, B, V), jnp.float32),
    mesh=plsc.VectorSubcoreMesh(core_axis_name="c", subcore_axis_name="s"),
    compiler_params=pltpu.CompilerParams(
        needs_layout_passes=False, use_tc_tiling_on_sc=False),
    scratch_types=[
        pltpu.VMEM((NJ, G), jnp.int32),
        pltpu.VMEM((NJ, G), jnp.int32),
        _PLANE, _PLANE, _PLANE, _PLANE,
        pltpu.VMEM((NP, W), jnp.float32),
        pltpu.VMEM((NP, 4, W), jnp.float32),
        _FLATP, _FLATP, _FLATP, _FLATP, _FLATP,
    ] + [pltpu.SemaphoreType.DMA] * (2 * NP),
)
def _apply_deltas_sc(scores_hbm, deltas_hbm, anch_hbm, idx_hbm, out_hbm,
                     *refs):
    _sc_body(scores_hbm, deltas_hbm, anch_hbm, idx_hbm, out_hbm, *refs)


def kernel(scores, deltas, anchor_boxes, valid_indices):
    idx = valid_indices.astype(jnp.int32)
    deltas_t = jnp.transpose(deltas, (0, 2, 1)).reshape(B * 4, AB)
    anch_t = jnp.transpose(anchor_boxes, (1, 0))
    out = _apply_deltas_sc(scores, deltas_t, anch_t, idx)
    return jnp.transpose(out, (1, 2, 0))
